# Initial kernel scaffold; baseline (speedup 1.0000x reference)
import jax
import jax.numpy as jnp
from jax.experimental import pallas as pl


def kernel(drug_drug_batch, x, edge_index, batch, W1, b1, W2, b2, RW1, Rb1, RW2, Rb2):
    raise NotImplementedError("write your pallas kernel here")


# speedup vs baseline: 7.9206x; 7.9206x over previous
import functools

import jax
import jax.numpy as jnp
from jax import lax
from jax.experimental import pallas as pl
from jax.experimental.pallas import tpu as pltpu
from jax.experimental.pallas import tpu_sc as plsc

_N = 10000
_E = 320000
_G = 500
_D = 128
_DH = 256
_P = 4096

_NC, _NS, _L = 2, 16, 16
_NW = _NC * _NS
_NP = 10240
_RPT = _NP // _NS
_CH = 128
_NCHUNK = 79
_SCHUNK = 158
_EPAD = _CH * _NCHUNK * _NW
_NH = _NP // 2
_ACCR = 5248
_GP = 512
_PPT = (2 * _P) // _NW
_NBLK = 16

_mesh = plsc.VectorSubcoreMesh(
    core_axis_name="c", subcore_axis_name="s", num_cores=_NC, num_subcores=_NS)

_f32 = jnp.float32
_i32 = jnp.int32


def _hist_body(dst_hbm, deg_out, eidx, hist):
    cid = lax.axis_index("c")
    sid = lax.axis_index("s")
    wid = sid * _NC + cid
    pltpu.sync_copy(dst_hbm.at[sid, cid], eidx)

    zero16 = jnp.zeros((_L,), _f32)

    def _z(i, _):
        hist[pl.ds(i * _L, _L)] = zero16
        return 0
    lax.fori_loop(0, _NP // _L, _z, 0)

    one16 = jnp.ones((_L,), _f32)

    def _edge(c, _):
        for j in range(_CH // _L):
            ids = eidx[c, pl.ds(j * _L, _L)]
            plsc.addupdate_scatter(hist, [ids], one16)
        return 0
    lax.fori_loop(0, _NCHUNK, _edge, 0)
    pltpu.sync_copy(hist, deg_out.at[wid])


_sc_hist = functools.partial(
    pl.kernel,
    out_type=jax.ShapeDtypeStruct((_NW, _NP), _f32),
    mesh=_mesh,
    compiler_params=pltpu.CompilerParams(needs_layout_passes=False),
    scratch_types=[pltpu.VMEM((_NCHUNK, _CH), _i32),
                   pltpu.VMEM((_NP,), _f32)],
)(_hist_body)


def _scatter_body(g_hbm, src_hbm, dst_hbm, part_out,
                  sidx, didx, rows, acc, semg, sems):
    cid = lax.axis_index("c")
    sid = lax.axis_index("s")
    pltpu.sync_copy(src_hbm.at[sid], sidx)
    pltpu.sync_copy(dst_hbm.at[sid], didx)

    base = cid * _NH

    def _remap(c, _):
        for j in range(_CH // _L):
            v = didx[c, pl.ds(j * _L, _L)] - base
            ok = (v >= 0) & (v < _NH)
            didx[c, pl.ds(j * _L, _L)] = jnp.where(ok, v, _NH)
        return 0
    lax.fori_loop(0, _SCHUNK, _remap, 0)

    zero16 = jnp.zeros((_L,), _f32)

    def _zrow(i, _):
        for j in range(_D // _L):
            rows[0, i, pl.ds(j * _L, _L)] = zero16
        return 0
    lax.fori_loop(0, _CH, _zrow, 0)
    zslab = _ACCR // _NS
    pltpu.sync_copy(rows.at[0], acc.at[pl.ds(sid * zslab, _CH)])
    pltpu.sync_copy(rows.at[0], acc.at[pl.ds(sid * zslab + _CH, _CH)])
    pltpu.sync_copy(rows.at[0, pl.ds(0, zslab - 2 * _CH)],
                    acc.at[pl.ds(sid * zslab + 2 * _CH, zslab - 2 * _CH)])
    plsc.subcore_barrier()

    def _step(c, _):
        pltpu.async_copy(g_hbm.at[sidx.at[c]], rows.at[0], semg).wait()
        pltpu.sync_copy(rows.at[0], acc.at[didx.at[c]], add=True)
        return 0
    lax.fori_loop(0, _SCHUNK, _step, 0)
    plsc.subcore_barrier()

    slab = _NH // _NS
    pltpu.sync_copy(acc.at[pl.ds(sid * slab, slab)],
                    part_out.at[pl.ds(cid * _NH + sid * slab, slab)])


_sc_scatter = functools.partial(
    pl.kernel,
    out_type=jax.ShapeDtypeStruct((_NP, _D), _f32),
    mesh=_mesh,
    scratch_types=[pltpu.VMEM((_SCHUNK, _CH), _i32),
                   pltpu.VMEM((_SCHUNK, _CH), _i32),
                   pltpu.VMEM((2, _CH, _D), _f32),
                   pltpu.VMEM_SHARED((_ACCR, _D), _f32),
                   pltpu.SemaphoreType.DMA,
                   pltpu.SemaphoreType.DMA],
)(_scatter_body)


def _pair_body(ge_hbm, qtz_hbm, cat_out, qidx, rows, sem):
    cid = lax.axis_index("c")
    sid = lax.axis_index("s")
    wid = sid * _NC + cid
    pltpu.sync_copy(qtz_hbm.at[wid], qidx)
    for half in range(2):
        pltpu.async_copy(ge_hbm.at[qidx.at[half]],
                         rows.at[pl.ds(half * _CH, _CH)], sem).wait()
    pltpu.sync_copy(rows, cat_out.at[pl.ds(wid * _PPT, _PPT)])


_sc_pair = functools.partial(
    pl.kernel,
    out_type=jax.ShapeDtypeStruct((2 * _P, _D), _f32),
    mesh=_mesh,
    scratch_types=[pltpu.VMEM((2, _CH), _i32),
                   pltpu.VMEM((_PPT, _D), _f32),
                   pltpu.SemaphoreType.DMA],
)(_pair_body)


def _dinv_of(deg_blk):
    d = 1.0 + jnp.sum(deg_blk, axis=1, keepdims=True)
    return lax.rsqrt(jnp.maximum(d, 1.0))


def _mm_body(x_ref, w_ref, o_ref):
    o_ref[...] = jnp.dot(x_ref[...], w_ref[...],
                         preferred_element_type=_f32,
                         precision=lax.Precision.HIGHEST)


def _tc_matmul(xp, w):
    return pl.pallas_call(
        _mm_body,
        grid=(_NBLK,),
        in_specs=[pl.BlockSpec((_RPT, _D), lambda i: (i, 0)),
                  pl.BlockSpec((_D, _D), lambda i: (0, 0))],
        out_specs=pl.BlockSpec((_RPT, _D), lambda i: (i, 0)),
        out_shape=jax.ShapeDtypeStruct((_NP, _D), _f32),
    )(xp, w)


def _scale_body(h_ref, deg_ref, g_ref):
    dinv = _dinv_of(deg_ref[...])
    g_ref[...] = h_ref[...] * dinv


def _tc_scale(h1, deg_part):
    return pl.pallas_call(
        _scale_body,
        grid=(_NBLK,),
        in_specs=[pl.BlockSpec((_RPT, _D), lambda i: (i, 0)),
                  pl.BlockSpec((_RPT, _NW), lambda i: (i, 0))],
        out_specs=pl.BlockSpec((_RPT, _D), lambda i: (i, 0)),
        out_shape=jax.ShapeDtypeStruct((_NP, _D), _f32),
    )(h1, deg_part)


def _pres_body(d_ref, pres_ref, acc):
    i = pl.program_id(0)
    @pl.when(i == 0)
    def _():
        acc[...] = jnp.zeros_like(acc)
    d = d_ref[...][0]
    gids = lax.broadcasted_iota(_i32, (1024, _GP), 1)
    oh = (gids == d).astype(_f32)
    acc[...] += jnp.dot(jnp.ones((1, 1024), _f32), oh,
                        preferred_element_type=_f32,
                        precision=lax.Precision.HIGHEST)
    @pl.when(i == 7)
    def _():
        pres_ref[...] = jnp.minimum(acc[...], 1.0)


def _tc_pres(ddb3):
    return pl.pallas_call(
        _pres_body,
        grid=(8,),
        in_specs=[pl.BlockSpec((1, 1024, 1), lambda i: (i, 0, 0))],
        out_specs=pl.BlockSpec((1, _GP), lambda i: (0, 0)),
        out_shape=jax.ShapeDtypeStruct((1, _GP), _f32),
        scratch_shapes=[pltpu.VMEM((1, _GP), _f32)],
    )(ddb3)


def _qtz_body(d_ref, pres_ref, q_ref):
    d = d_ref[...][0]
    gids = lax.broadcasted_iota(_i32, (_GP, 1024), 0)
    lt = (gids < d).astype(_f32)
    q = jnp.dot(pres_ref[...], lt, preferred_element_type=_f32,
                precision=lax.Precision.HIGHEST)
    q_ref[...] = jnp.round(q).astype(_i32).reshape(1, 1, 1024)


def _tc_qtz(ddb2, pres_row):
    return pl.pallas_call(
        _qtz_body,
        grid=(8,),
        in_specs=[pl.BlockSpec((1, 1, 1024), lambda i: (i, 0, 0)),
                  pl.BlockSpec((1, _GP), lambda i: (0, 0))],
        out_specs=pl.BlockSpec((1, 1, 1024), lambda i: (i, 0, 0)),
        out_shape=jax.ShapeDtypeStruct((8, 1, 1024), _i32),
    )(ddb2, pres_row)


def _comb1_body(p_ref, deg_ref, g_ref, b_ref, w_ref, o_ref):
    dinv = _dinv_of(deg_ref[...])
    t = dinv * (p_ref[...] + g_ref[...]) + b_ref[...]
    t = jnp.maximum(t, 0.0)
    ridx = (pl.program_id(0) * _RPT
            + lax.broadcasted_iota(_i32, (_RPT, 1), 0))
    vmask = (ridx < _N).astype(_f32)
    o_ref[...] = vmask * dinv * jnp.dot(t, w_ref[...],
                                        preferred_element_type=_f32,
                                        precision=lax.Precision.HIGHEST)


def _tc_combine1(part, deg_part, g1, b1, w2t):
    return pl.pallas_call(
        _comb1_body,
        grid=(_NBLK,),
        in_specs=[pl.BlockSpec((_RPT, _D), lambda i: (i, 0)),
                  pl.BlockSpec((_RPT, _NW), lambda i: (i, 0)),
                  pl.BlockSpec((_RPT, _D), lambda i: (i, 0)),
                  pl.BlockSpec((_D,), lambda i: (0,)),
                  pl.BlockSpec((_D, _D), lambda i: (0, 0))],
        out_specs=pl.BlockSpec((_RPT, _D), lambda i: (i, 0)),
        out_shape=jax.ShapeDtypeStruct((_NP, _D), _f32),
    )(part, deg_part, g1, b1, w2t)


def _comb2_body(p_ref, deg_ref, g_ref, b_ref, bat_ref, ge_ref, sums, cnt):
    i = pl.program_id(0)
    @pl.when(i == 0)
    def _():
        sums[...] = jnp.zeros_like(sums)
        cnt[...] = jnp.zeros_like(cnt)
    dinv = _dinv_of(deg_ref[...])
    t = jnp.maximum(dinv * (p_ref[...] + g_ref[...]) + b_ref[...], 0.0)
    bat = bat_ref[...][0]
    gr = lax.broadcasted_iota(_i32, (_GP, _RPT), 0)
    oht = (gr == bat).astype(_f32)
    hi = lax.Precision.HIGHEST
    sums[...] += jnp.dot(oht, t, preferred_element_type=_f32, precision=hi)
    cnt[...] += jnp.dot(oht, jnp.ones((_RPT, 1), _f32),
                        preferred_element_type=_f32, precision=hi)
    @pl.when(i == _NBLK - 1)
    def _():
        ge_ref[...] = sums[...] / jnp.maximum(cnt[...], 1.0)


def _tc_combine2(part, deg_part, g2, b2, batch2d):
    return pl.pallas_call(
        _comb2_body,
        grid=(_NBLK,),
        in_specs=[pl.BlockSpec((_RPT, _D), lambda i: (i, 0)),
                  pl.BlockSpec((_RPT, _NW), lambda i: (i, 0)),
                  pl.BlockSpec((_RPT, _D), lambda i: (i, 0)),
                  pl.BlockSpec((_D,), lambda i: (0,)),
                  pl.BlockSpec((1, 1, _RPT), lambda i: (i, 0, 0))],
        out_specs=pl.BlockSpec((_GP, _D), lambda i: (0, 0)),
        out_shape=jax.ShapeDtypeStruct((_GP, _D), _f32),
        scratch_shapes=[pltpu.VMEM((_GP, _D), _f32),
                        pltpu.VMEM((_GP, 1), _f32)],
    )(part, deg_part, g2, b2, batch2d)


def _mlp_body(a0, a1, wa, wb, rb1, rw2, rb2, o_ref):
    hi = lax.Precision.HIGHEST
    z = (jnp.dot(a0[...], wa[...], preferred_element_type=_f32, precision=hi)
         + jnp.dot(a1[...], wb[...], preferred_element_type=_f32, precision=hi)
         + rb1[...])
    z = jnp.maximum(z, 0.0)
    o_ref[...] = jnp.sum(z * rw2[...], axis=1, keepdims=True) + rb2[0]


def _tc_mlp(a0, a1, w1at, w1bt, rb1, rw2, rb2):
    blk = 512
    return pl.pallas_call(
        _mlp_body,
        grid=(_P // blk,),
        in_specs=[pl.BlockSpec((blk, _D), lambda i: (i, 0)),
                  pl.BlockSpec((blk, _D), lambda i: (i, 0)),
                  pl.BlockSpec((_D, _DH), lambda i: (0, 0)),
                  pl.BlockSpec((_D, _DH), lambda i: (0, 0)),
                  pl.BlockSpec((_DH,), lambda i: (0,)),
                  pl.BlockSpec((1, _DH), lambda i: (0, 0)),
                  pl.BlockSpec(memory_space=pltpu.SMEM)],
        out_specs=pl.BlockSpec((blk, 1), lambda i: (i, 0)),
        out_shape=jax.ShapeDtypeStruct((_P, 1), _f32),
    )(a0, a1, w1at, w1bt, rb1, rw2, rb2)


def kernel(drug_drug_batch, x, edge_index, batch,
           W1, b1, W2, b2, RW1, Rb1, RW2, Rb2):
    src, dst = edge_index[0], edge_index[1]
    pad_e = _EPAD - _E
    fill = jnp.full((pad_e,), _N, _i32)
    srcp = jnp.concatenate([src, fill]).reshape(_NS, _SCHUNK, _CH)
    dstp = jnp.concatenate([dst, fill]).reshape(_NS, _SCHUNK, _CH)
    xp = jnp.pad(x, ((0, _NP - _N), (0, 0)))
    batch2d = jnp.concatenate(
        [batch, jnp.full((_NP - _N,), _G + 5, _i32)]).reshape(_NBLK, 1, _RPT)
    flat = drug_drug_batch.reshape(2 * _P)
    w1t, w2t = W1.T, W2.T
    w1at, w1bt = RW1[:, :_D].T, RW1[:, _D:].T

    deg_part = _sc_hist(dstp.reshape(_NS, _NC, _NCHUNK, _CH)).T
    pres_row = _tc_pres(flat.reshape(8, 1024, 1))
    qtz = _tc_qtz(flat.reshape(8, 1, 1024), pres_row)
    h1 = _tc_matmul(xp, w1t)
    g1 = _tc_scale(h1, deg_part)
    part1 = _sc_scatter(g1, srcp, dstp)
    g2 = _tc_combine1(part1, deg_part, g1, b1, w2t)
    part2 = _sc_scatter(g2, srcp, dstp)
    ge = _tc_combine2(part2, deg_part, g2, b2, batch2d)
    cat2 = _sc_pair(ge, qtz.reshape(_NW, 2, _CH))
    return _tc_mlp(cat2[:_P], cat2[_P:], w1at, w1bt, Rb1, RW2, Rb2)


# speedup vs baseline: 8.7832x; 1.1089x over previous
import functools

import jax
import jax.numpy as jnp
from jax import lax
from jax.experimental import pallas as pl
from jax.experimental.pallas import tpu as pltpu
from jax.experimental.pallas import tpu_sc as plsc

_N = 10000
_E = 320000
_G = 500
_D = 128
_DH = 256
_P = 4096

_NC, _NS, _L = 2, 16, 16
_NW = _NC * _NS
_NP = 10240
_RPT = _NP // _NS
_CH = 128
_NCHUNK = 79
_SCHUNK = 158
_EPAD = _CH * _NCHUNK * _NW
_NH = _NP // 2
_ACCR = 5248
_GP = 512
_NBUF = 3
_PPT = (2 * _P) // _NW
_NBLK = 16

_mesh = plsc.VectorSubcoreMesh(
    core_axis_name="c", subcore_axis_name="s", num_cores=_NC, num_subcores=_NS)

_f32 = jnp.float32
_i32 = jnp.int32


def _hist_body(dst_hbm, deg_out, eidx, hist):
    cid = lax.axis_index("c")
    sid = lax.axis_index("s")
    wid = sid * _NC + cid
    pltpu.sync_copy(dst_hbm.at[wid], eidx)

    zero16 = jnp.zeros((_L,), _f32)

    def _z(i, _):
        hist[pl.ds(i * _L, _L)] = zero16
        return 0
    lax.fori_loop(0, _NP // _L, _z, 0)

    one16 = jnp.ones((_L,), _f32)

    def _edge(c, _):
        for j in range(_CH // _L):
            ids = eidx[c, pl.ds(j * _L, _L)]
            plsc.addupdate_scatter(hist, [ids], one16)
        return 0
    lax.fori_loop(0, _NCHUNK, _edge, 0)
    pltpu.sync_copy(hist, deg_out.at[wid])


_sc_hist = functools.partial(
    pl.kernel,
    out_type=jax.ShapeDtypeStruct((_NW, _NP), _f32),
    mesh=_mesh,
    compiler_params=pltpu.CompilerParams(needs_layout_passes=False),
    scratch_types=[pltpu.VMEM((_NCHUNK, _CH), _i32),
                   pltpu.VMEM((_NP,), _f32)],
)(_hist_body)


def _scatter_body(g_hbm, src_hbm, dst_hbm, part_out,
                  sidx, didx, rows, acc, semg, sems):
    cid = lax.axis_index("c")
    sid = lax.axis_index("s")
    base = cid * _NH
    zero16 = jnp.zeros((_L,), _f32)

    def _zrow(i, _):
        for j in range(_D // _L):
            rows[0, i, pl.ds(j * _L, _L)] = zero16
        return 0
    lax.fori_loop(0, _CH, _zrow, 0)
    zslab = _ACCR // _NS
    pltpu.sync_copy(rows.at[0], acc.at[pl.ds(sid * zslab, _CH)])
    pltpu.sync_copy(rows.at[0], acc.at[pl.ds(sid * zslab + _CH, _CH)])
    pltpu.sync_copy(rows.at[0, pl.ds(0, zslab - 2 * _CH)],
                    acc.at[pl.ds(sid * zslab + 2 * _CH, zslab - 2 * _CH)])
    plsc.subcore_barrier()

    for h in range(2):
        hh = jnp.int32(h) + 0 * cid
        pltpu.sync_copy(src_hbm.at[sid, hh], sidx)
        pltpu.sync_copy(dst_hbm.at[sid, hh], didx)

        def _remap(c, _):
            for j in range(_CH // _L):
                v = didx[c, pl.ds(j * _L, _L)] - base
                ok = (v >= 0) & (v < _NH)
                didx[c, pl.ds(j * _L, _L)] = jnp.where(ok, v, _NH)
            return 0
        lax.fori_loop(0, _NCHUNK, _remap, 0)

        pltpu.async_copy(g_hbm.at[sidx.at[0]], rows.at[0], semg)
        pltpu.async_copy(g_hbm.at[sidx.at[1]], rows.at[1], semg)

        def _step(c, _):
            b = lax.rem(c, _NBUF)
            pltpu.make_async_copy(g_hbm.at[pl.ds(0, _CH)], rows.at[b],
                                  semg).wait()
            pltpu.async_copy(rows.at[b], acc.at[didx.at[c]], sems, add=True)
            @pl.when(c >= 1)
            def _():
                bo = lax.rem(c - 1, _NBUF)
                pltpu.make_async_copy(rows.at[bo], acc.at[pl.ds(0, _CH)],
                                      sems).wait()
            @pl.when(c + 2 < _NCHUNK)
            def _():
                bn = lax.rem(c + 2, _NBUF)
                pltpu.async_copy(g_hbm.at[sidx.at[c + 2]], rows.at[bn], semg)
            return 0
        lax.fori_loop(0, _NCHUNK, _step, 0)
        pltpu.make_async_copy(rows.at[(_NCHUNK - 1) % _NBUF],
                              acc.at[pl.ds(0, _CH)], sems).wait()
    plsc.subcore_barrier()

    slab = _NH // _NS
    pltpu.sync_copy(acc.at[pl.ds(sid * slab, slab)],
                    part_out.at[pl.ds(cid * _NH + sid * slab, slab)])


_sc_scatter = functools.partial(
    pl.kernel,
    out_type=jax.ShapeDtypeStruct((_NP, _D), _f32),
    mesh=_mesh,
    compiler_params=pltpu.CompilerParams(needs_layout_passes=False),
    scratch_types=[pltpu.VMEM((_NCHUNK, _CH), _i32),
                   pltpu.VMEM((_NCHUNK, _CH), _i32),
                   pltpu.VMEM((_NBUF, _CH, _D), _f32),
                   pltpu.VMEM_SHARED((_ACCR, _D), _f32),
                   pltpu.SemaphoreType.DMA,
                   pltpu.SemaphoreType.DMA],
)(_scatter_body)


def _pair_body(ge_hbm, qtz_hbm, cat_out, qidx, rows, sem):
    cid = lax.axis_index("c")
    sid = lax.axis_index("s")
    wid = sid * _NC + cid
    pltpu.sync_copy(qtz_hbm.at[wid], qidx)
    for half in range(2):
        pltpu.async_copy(ge_hbm.at[qidx.at[half]],
                         rows.at[pl.ds(half * _CH, _CH)], sem).wait()
    pltpu.sync_copy(rows, cat_out.at[pl.ds(wid * _PPT, _PPT)])


_sc_pair = functools.partial(
    pl.kernel,
    out_type=jax.ShapeDtypeStruct((2 * _P, _D), _f32),
    mesh=_mesh,
    scratch_types=[pltpu.VMEM((2, _CH), _i32),
                   pltpu.VMEM((_PPT, _D), _f32),
                   pltpu.SemaphoreType.DMA],
)(_pair_body)


def _dinv_of(deg_blk):
    d = 1.0 + jnp.sum(deg_blk, axis=1, keepdims=True)
    return lax.rsqrt(jnp.maximum(d, 1.0))


def _mm_body(x_ref, w_ref, o_ref):
    o_ref[...] = jnp.dot(x_ref[...], w_ref[...],
                         preferred_element_type=_f32,
                         precision=lax.Precision.HIGHEST)


def _tc_matmul(xp, w):
    return pl.pallas_call(
        _mm_body,
        grid=(_NBLK,),
        in_specs=[pl.BlockSpec((_RPT, _D), lambda i: (i, 0)),
                  pl.BlockSpec((_D, _D), lambda i: (0, 0))],
        out_specs=pl.BlockSpec((_RPT, _D), lambda i: (i, 0)),
        out_shape=jax.ShapeDtypeStruct((_NP, _D), _f32),
    )(xp, w)


def _scale_body(h_ref, deg_ref, g_ref):
    dinv = _dinv_of(deg_ref[...])
    g_ref[...] = h_ref[...] * dinv


def _tc_scale(h1, deg_part):
    return pl.pallas_call(
        _scale_body,
        grid=(_NBLK,),
        in_specs=[pl.BlockSpec((_RPT, _D), lambda i: (i, 0)),
                  pl.BlockSpec((_RPT, _NW), lambda i: (i, 0))],
        out_specs=pl.BlockSpec((_RPT, _D), lambda i: (i, 0)),
        out_shape=jax.ShapeDtypeStruct((_NP, _D), _f32),
    )(h1, deg_part)


def _pres_body(d_ref, pres_ref, acc):
    i = pl.program_id(0)
    @pl.when(i == 0)
    def _():
        acc[...] = jnp.zeros_like(acc)
    d = d_ref[...][0]
    gids = lax.broadcasted_iota(_i32, (1024, _GP), 1)
    oh = (gids == d).astype(_f32)
    acc[...] += jnp.dot(jnp.ones((1, 1024), _f32), oh,
                        preferred_element_type=_f32,
                        precision=lax.Precision.HIGHEST)
    @pl.when(i == 7)
    def _():
        pres_ref[...] = jnp.minimum(acc[...], 1.0)


def _tc_pres(ddb3):
    return pl.pallas_call(
        _pres_body,
        grid=(8,),
        in_specs=[pl.BlockSpec((1, 1024, 1), lambda i: (i, 0, 0))],
        out_specs=pl.BlockSpec((1, _GP), lambda i: (0, 0)),
        out_shape=jax.ShapeDtypeStruct((1, _GP), _f32),
        scratch_shapes=[pltpu.VMEM((1, _GP), _f32)],
    )(ddb3)


def _qtz_body(d_ref, pres_ref, q_ref):
    d = d_ref[...][0]
    gids = lax.broadcasted_iota(_i32, (_GP, 1024), 0)
    lt = (gids < d).astype(_f32)
    q = jnp.dot(pres_ref[...], lt, preferred_element_type=_f32,
                precision=lax.Precision.HIGHEST)
    q_ref[...] = jnp.round(q).astype(_i32).reshape(1, 1, 1024)


def _tc_qtz(ddb2, pres_row):
    return pl.pallas_call(
        _qtz_body,
        grid=(8,),
        in_specs=[pl.BlockSpec((1, 1, 1024), lambda i: (i, 0, 0)),
                  pl.BlockSpec((1, _GP), lambda i: (0, 0))],
        out_specs=pl.BlockSpec((1, 1, 1024), lambda i: (i, 0, 0)),
        out_shape=jax.ShapeDtypeStruct((8, 1, 1024), _i32),
    )(ddb2, pres_row)


def _comb1_body(p_ref, deg_ref, g_ref, b_ref, w_ref, o_ref):
    dinv = _dinv_of(deg_ref[...])
    t = dinv * (p_ref[...] + g_ref[...]) + b_ref[...]
    t = jnp.maximum(t, 0.0)
    ridx = (pl.program_id(0) * _RPT
            + lax.broadcasted_iota(_i32, (_RPT, 1), 0))
    vmask = (ridx < _N).astype(_f32)
    o_ref[...] = vmask * dinv * jnp.dot(t, w_ref[...],
                                        preferred_element_type=_f32,
                                        precision=lax.Precision.HIGHEST)


def _tc_combine1(part, deg_part, g1, b1, w2t):
    return pl.pallas_call(
        _comb1_body,
        grid=(_NBLK,),
        in_specs=[pl.BlockSpec((_RPT, _D), lambda i: (i, 0)),
                  pl.BlockSpec((_RPT, _NW), lambda i: (i, 0)),
                  pl.BlockSpec((_RPT, _D), lambda i: (i, 0)),
                  pl.BlockSpec((_D,), lambda i: (0,)),
                  pl.BlockSpec((_D, _D), lambda i: (0, 0))],
        out_specs=pl.BlockSpec((_RPT, _D), lambda i: (i, 0)),
        out_shape=jax.ShapeDtypeStruct((_NP, _D), _f32),
    )(part, deg_part, g1, b1, w2t)


def _comb2_body(p_ref, deg_ref, g_ref, b_ref, bat_ref, ge_ref, sums, cnt):
    i = pl.program_id(0)
    @pl.when(i == 0)
    def _():
        sums[...] = jnp.zeros_like(sums)
        cnt[...] = jnp.zeros_like(cnt)
    dinv = _dinv_of(deg_ref[...])
    t = jnp.maximum(dinv * (p_ref[...] + g_ref[...]) + b_ref[...], 0.0)
    bat = bat_ref[...][0]
    gr = lax.broadcasted_iota(_i32, (_GP, _RPT), 0)
    oht = (gr == bat).astype(_f32)
    hi = lax.Precision.HIGHEST
    sums[...] += jnp.dot(oht, t, preferred_element_type=_f32, precision=hi)
    cnt[...] += jnp.dot(oht, jnp.ones((_RPT, 1), _f32),
                        preferred_element_type=_f32, precision=hi)
    @pl.when(i == _NBLK - 1)
    def _():
        ge_ref[...] = sums[...] / jnp.maximum(cnt[...], 1.0)


def _tc_combine2(part, deg_part, g2, b2, batch2d):
    return pl.pallas_call(
        _comb2_body,
        grid=(_NBLK,),
        in_specs=[pl.BlockSpec((_RPT, _D), lambda i: (i, 0)),
                  pl.BlockSpec((_RPT, _NW), lambda i: (i, 0)),
                  pl.BlockSpec((_RPT, _D), lambda i: (i, 0)),
                  pl.BlockSpec((_D,), lambda i: (0,)),
                  pl.BlockSpec((1, 1, _RPT), lambda i: (i, 0, 0))],
        out_specs=pl.BlockSpec((_GP, _D), lambda i: (0, 0)),
        out_shape=jax.ShapeDtypeStruct((_GP, _D), _f32),
        scratch_shapes=[pltpu.VMEM((_GP, _D), _f32),
                        pltpu.VMEM((_GP, 1), _f32)],
    )(part, deg_part, g2, b2, batch2d)


def _mlp_body(a0, a1, wa, wb, rb1, rw2, rb2, o_ref):
    hi = lax.Precision.HIGHEST
    z = (jnp.dot(a0[...], wa[...], preferred_element_type=_f32, precision=hi)
         + jnp.dot(a1[...], wb[...], preferred_element_type=_f32, precision=hi)
         + rb1[...])
    z = jnp.maximum(z, 0.0)
    o_ref[...] = jnp.sum(z * rw2[...], axis=1, keepdims=True) + rb2[0]


def _tc_mlp(a0, a1, w1at, w1bt, rb1, rw2, rb2):
    blk = 512
    return pl.pallas_call(
        _mlp_body,
        grid=(_P // blk,),
        in_specs=[pl.BlockSpec((blk, _D), lambda i: (i, 0)),
                  pl.BlockSpec((blk, _D), lambda i: (i, 0)),
                  pl.BlockSpec((_D, _DH), lambda i: (0, 0)),
                  pl.BlockSpec((_D, _DH), lambda i: (0, 0)),
                  pl.BlockSpec((_DH,), lambda i: (0,)),
                  pl.BlockSpec((1, _DH), lambda i: (0, 0)),
                  pl.BlockSpec(memory_space=pltpu.SMEM)],
        out_specs=pl.BlockSpec((blk, 1), lambda i: (i, 0)),
        out_shape=jax.ShapeDtypeStruct((_P, 1), _f32),
    )(a0, a1, w1at, w1bt, rb1, rw2, rb2)


def kernel(drug_drug_batch, x, edge_index, batch,
           W1, b1, W2, b2, RW1, Rb1, RW2, Rb2):
    src, dst = edge_index[0], edge_index[1]
    pad_e = _EPAD - _E
    fill = jnp.full((pad_e,), _N, _i32)
    srcp = jnp.concatenate([src, fill]).reshape(_NS, _NC, _NCHUNK, _CH)
    dstp = jnp.concatenate([dst, fill]).reshape(_NS, _NC, _NCHUNK, _CH)
    xp = jnp.pad(x, ((0, _NP - _N), (0, 0)))
    batch2d = jnp.concatenate(
        [batch, jnp.full((_NP - _N,), _G + 5, _i32)]).reshape(_NBLK, 1, _RPT)
    flat = drug_drug_batch.reshape(2 * _P)
    w1t, w2t = W1.T, W2.T
    w1at, w1bt = RW1[:, :_D].T, RW1[:, _D:].T

    deg_part = _sc_hist(dstp.reshape(_NW, _NCHUNK, _CH)).T
    pres_row = _tc_pres(flat.reshape(8, 1024, 1))
    qtz = _tc_qtz(flat.reshape(8, 1, 1024), pres_row)
    h1 = _tc_matmul(xp, w1t)
    g1 = _tc_scale(h1, deg_part)
    part1 = _sc_scatter(g1, srcp, dstp)
    g2 = _tc_combine1(part1, deg_part, g1, b1, w2t)
    part2 = _sc_scatter(g2, srcp, dstp)
    ge = _tc_combine2(part2, deg_part, g2, b2, batch2d)
    cat2 = _sc_pair(ge, qtz.reshape(_NW, 2, _CH))
    return _tc_mlp(cat2[:_P], cat2[_P:], w1at, w1bt, Rb1, RW2, Rb2)


# speedup vs baseline: 8.7990x; 1.0018x over previous
import functools

import jax
import jax.numpy as jnp
from jax import lax
from jax.experimental import pallas as pl
from jax.experimental.pallas import tpu as pltpu
from jax.experimental.pallas import tpu_sc as plsc

_N = 10000
_E = 320000
_G = 500
_D = 128
_DH = 256
_P = 4096

_NC, _NS, _L = 2, 16, 16
_NW = _NC * _NS
_NP = 10240
_RPT = _NP // _NS
_CH = 128
_NCHUNK = 79
_SCHUNK = 158
_EPAD = _CH * _NCHUNK * _NW
_NH = _NP // 2
_ACCR = 5248
_GP = 512
_NBUF = 4
_PPT = (2 * _P) // _NW
_NBLK = 16

_mesh = plsc.VectorSubcoreMesh(
    core_axis_name="c", subcore_axis_name="s", num_cores=_NC, num_subcores=_NS)

_f32 = jnp.float32
_i32 = jnp.int32


def _hist_body(dst_hbm, deg_out, eidx, hist):
    cid = lax.axis_index("c")
    sid = lax.axis_index("s")
    wid = sid * _NC + cid
    pltpu.sync_copy(dst_hbm.at[wid], eidx)

    zero16 = jnp.zeros((_L,), _f32)

    def _z(i, _):
        hist[pl.ds(i * _L, _L)] = zero16
        return 0
    lax.fori_loop(0, _NP // _L, _z, 0)

    one16 = jnp.ones((_L,), _f32)

    def _edge(c, _):
        for j in range(_CH // _L):
            ids = eidx[c, pl.ds(j * _L, _L)]
            plsc.addupdate_scatter(hist, [ids], one16)
        return 0
    lax.fori_loop(0, _NCHUNK, _edge, 0)
    pltpu.sync_copy(hist, deg_out.at[wid])


_sc_hist = functools.partial(
    pl.kernel,
    out_type=jax.ShapeDtypeStruct((_NW, _NP), _f32),
    mesh=_mesh,
    compiler_params=pltpu.CompilerParams(needs_layout_passes=False),
    scratch_types=[pltpu.VMEM((_NCHUNK, _CH), _i32),
                   pltpu.VMEM((_NP,), _f32)],
)(_hist_body)


def _scatter_body(g_hbm, src_hbm, dst_hbm, part_out,
                  sidx, didx, rows, acc, semg, sems):
    cid = lax.axis_index("c")
    sid = lax.axis_index("s")
    base = cid * _NH
    zero16 = jnp.zeros((_L,), _f32)

    def _zrow(i, _):
        for j in range(_D // _L):
            rows[0, i, pl.ds(j * _L, _L)] = zero16
        return 0
    lax.fori_loop(0, _CH, _zrow, 0)
    zslab = _ACCR // _NS
    pltpu.sync_copy(rows.at[0], acc.at[pl.ds(sid * zslab, _CH)])
    pltpu.sync_copy(rows.at[0], acc.at[pl.ds(sid * zslab + _CH, _CH)])
    pltpu.sync_copy(rows.at[0, pl.ds(0, zslab - 2 * _CH)],
                    acc.at[pl.ds(sid * zslab + 2 * _CH, zslab - 2 * _CH)])
    plsc.subcore_barrier()

    for h in range(2):
        hh = jnp.int32(h) + 0 * cid
        pltpu.sync_copy(src_hbm.at[sid, hh], sidx)
        pltpu.sync_copy(dst_hbm.at[sid, hh], didx)

        def _remap(c, _):
            for j in range(_CH // _L):
                v = didx[c, pl.ds(j * _L, _L)] - base
                ok = (v >= 0) & (v < _NH)
                didx[c, pl.ds(j * _L, _L)] = jnp.where(ok, v, _NH)
            return 0
        lax.fori_loop(0, _NCHUNK, _remap, 0)

        pltpu.async_copy(g_hbm.at[sidx.at[0]], rows.at[0], semg)
        pltpu.async_copy(g_hbm.at[sidx.at[1]], rows.at[1], semg)

        def _step(c, _):
            b = lax.rem(c, _NBUF)
            pltpu.make_async_copy(g_hbm.at[pl.ds(0, _CH)], rows.at[b],
                                  semg).wait()
            pltpu.async_copy(rows.at[b], acc.at[didx.at[c]], sems, add=True)
            @pl.when(c >= 2)
            def _():
                bo = lax.rem(c - 2, _NBUF)
                pltpu.make_async_copy(rows.at[bo], acc.at[pl.ds(0, _CH)],
                                      sems).wait()
            @pl.when(c + 2 < _NCHUNK)
            def _():
                bn = lax.rem(c + 2, _NBUF)
                pltpu.async_copy(g_hbm.at[sidx.at[c + 2]], rows.at[bn], semg)
            return 0
        lax.fori_loop(0, _NCHUNK, _step, 0)
        for _c in (_NCHUNK - 2, _NCHUNK - 1):
            pltpu.make_async_copy(rows.at[_c % _NBUF],
                                  acc.at[pl.ds(0, _CH)], sems).wait()
    plsc.subcore_barrier()

    slab = _NH // _NS
    pltpu.sync_copy(acc.at[pl.ds(sid * slab, slab)],
                    part_out.at[pl.ds(cid * _NH + sid * slab, slab)])


_sc_scatter = functools.partial(
    pl.kernel,
    out_type=jax.ShapeDtypeStruct((_NP, _D), _f32),
    mesh=_mesh,
    compiler_params=pltpu.CompilerParams(needs_layout_passes=False),
    scratch_types=[pltpu.VMEM((_NCHUNK, _CH), _i32),
                   pltpu.VMEM((_NCHUNK, _CH), _i32),
                   pltpu.VMEM((_NBUF, _CH, _D), _f32),
                   pltpu.VMEM_SHARED((_ACCR, _D), _f32),
                   pltpu.SemaphoreType.DMA,
                   pltpu.SemaphoreType.DMA],
)(_scatter_body)


def _pair_body(ge_hbm, qtz_hbm, cat_out, qidx, rows, sem):
    cid = lax.axis_index("c")
    sid = lax.axis_index("s")
    wid = sid * _NC + cid
    pltpu.sync_copy(qtz_hbm.at[wid], qidx)
    for half in range(2):
        pltpu.async_copy(ge_hbm.at[qidx.at[half]],
                         rows.at[pl.ds(half * _CH, _CH)], sem).wait()
    pltpu.sync_copy(rows, cat_out.at[pl.ds(wid * _PPT, _PPT)])


_sc_pair = functools.partial(
    pl.kernel,
    out_type=jax.ShapeDtypeStruct((2 * _P, _D), _f32),
    mesh=_mesh,
    scratch_types=[pltpu.VMEM((2, _CH), _i32),
                   pltpu.VMEM((_PPT, _D), _f32),
                   pltpu.SemaphoreType.DMA],
)(_pair_body)


def _dinv_of(deg_blk):
    d = 1.0 + jnp.sum(deg_blk, axis=1, keepdims=True)
    return lax.rsqrt(jnp.maximum(d, 1.0))


def _mm_body(x_ref, w_ref, o_ref):
    o_ref[...] = jnp.dot(x_ref[...], w_ref[...],
                         preferred_element_type=_f32,
                         precision=lax.Precision.HIGHEST)


def _tc_matmul(xp, w):
    return pl.pallas_call(
        _mm_body,
        grid=(_NBLK,),
        in_specs=[pl.BlockSpec((_RPT, _D), lambda i: (i, 0)),
                  pl.BlockSpec((_D, _D), lambda i: (0, 0))],
        out_specs=pl.BlockSpec((_RPT, _D), lambda i: (i, 0)),
        out_shape=jax.ShapeDtypeStruct((_NP, _D), _f32),
    )(xp, w)


def _scale_body(h_ref, deg_ref, g_ref):
    dinv = _dinv_of(deg_ref[...])
    g_ref[...] = h_ref[...] * dinv


def _tc_scale(h1, deg_part):
    return pl.pallas_call(
        _scale_body,
        grid=(_NBLK,),
        in_specs=[pl.BlockSpec((_RPT, _D), lambda i: (i, 0)),
                  pl.BlockSpec((_RPT, _NW), lambda i: (i, 0))],
        out_specs=pl.BlockSpec((_RPT, _D), lambda i: (i, 0)),
        out_shape=jax.ShapeDtypeStruct((_NP, _D), _f32),
    )(h1, deg_part)


def _pres_body(d_ref, pres_ref, acc):
    i = pl.program_id(0)
    @pl.when(i == 0)
    def _():
        acc[...] = jnp.zeros_like(acc)
    d = d_ref[...][0]
    gids = lax.broadcasted_iota(_i32, (1024, _GP), 1)
    oh = (gids == d).astype(_f32)
    acc[...] += jnp.dot(jnp.ones((1, 1024), _f32), oh,
                        preferred_element_type=_f32,
                        precision=lax.Precision.HIGHEST)
    @pl.when(i == 7)
    def _():
        pres_ref[...] = jnp.minimum(acc[...], 1.0)


def _tc_pres(ddb3):
    return pl.pallas_call(
        _pres_body,
        grid=(8,),
        in_specs=[pl.BlockSpec((1, 1024, 1), lambda i: (i, 0, 0))],
        out_specs=pl.BlockSpec((1, _GP), lambda i: (0, 0)),
        out_shape=jax.ShapeDtypeStruct((1, _GP), _f32),
        scratch_shapes=[pltpu.VMEM((1, _GP), _f32)],
    )(ddb3)


def _qtz_body(d_ref, pres_ref, q_ref):
    d = d_ref[...][0]
    gids = lax.broadcasted_iota(_i32, (_GP, 1024), 0)
    lt = (gids < d).astype(_f32)
    q = jnp.dot(pres_ref[...], lt, preferred_element_type=_f32,
                precision=lax.Precision.HIGHEST)
    q_ref[...] = jnp.round(q).astype(_i32).reshape(1, 1, 1024)


def _tc_qtz(ddb2, pres_row):
    return pl.pallas_call(
        _qtz_body,
        grid=(8,),
        in_specs=[pl.BlockSpec((1, 1, 1024), lambda i: (i, 0, 0)),
                  pl.BlockSpec((1, _GP), lambda i: (0, 0))],
        out_specs=pl.BlockSpec((1, 1, 1024), lambda i: (i, 0, 0)),
        out_shape=jax.ShapeDtypeStruct((8, 1, 1024), _i32),
    )(ddb2, pres_row)


def _comb1_body(p_ref, deg_ref, g_ref, b_ref, w_ref, o_ref):
    dinv = _dinv_of(deg_ref[...])
    t = dinv * (p_ref[...] + g_ref[...]) + b_ref[...]
    t = jnp.maximum(t, 0.0)
    ridx = (pl.program_id(0) * _RPT
            + lax.broadcasted_iota(_i32, (_RPT, 1), 0))
    vmask = (ridx < _N).astype(_f32)
    o_ref[...] = vmask * dinv * jnp.dot(t, w_ref[...],
                                        preferred_element_type=_f32,
                                        precision=lax.Precision.HIGHEST)


def _tc_combine1(part, deg_part, g1, b1, w2t):
    return pl.pallas_call(
        _comb1_body,
        grid=(_NBLK,),
        in_specs=[pl.BlockSpec((_RPT, _D), lambda i: (i, 0)),
                  pl.BlockSpec((_RPT, _NW), lambda i: (i, 0)),
                  pl.BlockSpec((_RPT, _D), lambda i: (i, 0)),
                  pl.BlockSpec((_D,), lambda i: (0,)),
                  pl.BlockSpec((_D, _D), lambda i: (0, 0))],
        out_specs=pl.BlockSpec((_RPT, _D), lambda i: (i, 0)),
        out_shape=jax.ShapeDtypeStruct((_NP, _D), _f32),
    )(part, deg_part, g1, b1, w2t)


def _comb2_body(p_ref, deg_ref, g_ref, b_ref, bat_ref, ge_ref, sums, cnt):
    i = pl.program_id(0)
    @pl.when(i == 0)
    def _():
        sums[...] = jnp.zeros_like(sums)
        cnt[...] = jnp.zeros_like(cnt)
    dinv = _dinv_of(deg_ref[...])
    t = jnp.maximum(dinv * (p_ref[...] + g_ref[...]) + b_ref[...], 0.0)
    bat = bat_ref[...][0]
    gr = lax.broadcasted_iota(_i32, (_GP, _RPT), 0)
    oht = (gr == bat).astype(_f32)
    hi = lax.Precision.HIGHEST
    sums[...] += jnp.dot(oht, t, preferred_element_type=_f32, precision=hi)
    cnt[...] += jnp.dot(oht, jnp.ones((_RPT, 1), _f32),
                        preferred_element_type=_f32, precision=hi)
    @pl.when(i == _NBLK - 1)
    def _():
        ge_ref[...] = sums[...] / jnp.maximum(cnt[...], 1.0)


def _tc_combine2(part, deg_part, g2, b2, batch2d):
    return pl.pallas_call(
        _comb2_body,
        grid=(_NBLK,),
        in_specs=[pl.BlockSpec((_RPT, _D), lambda i: (i, 0)),
                  pl.BlockSpec((_RPT, _NW), lambda i: (i, 0)),
                  pl.BlockSpec((_RPT, _D), lambda i: (i, 0)),
                  pl.BlockSpec((_D,), lambda i: (0,)),
                  pl.BlockSpec((1, 1, _RPT), lambda i: (i, 0, 0))],
        out_specs=pl.BlockSpec((_GP, _D), lambda i: (0, 0)),
        out_shape=jax.ShapeDtypeStruct((_GP, _D), _f32),
        scratch_shapes=[pltpu.VMEM((_GP, _D), _f32),
                        pltpu.VMEM((_GP, 1), _f32)],
    )(part, deg_part, g2, b2, batch2d)


def _mlp_body(a0, a1, wa, wb, rb1, rw2, rb2, o_ref):
    hi = lax.Precision.HIGHEST
    z = (jnp.dot(a0[...], wa[...], preferred_element_type=_f32, precision=hi)
         + jnp.dot(a1[...], wb[...], preferred_element_type=_f32, precision=hi)
         + rb1[...])
    z = jnp.maximum(z, 0.0)
    o_ref[...] = jnp.sum(z * rw2[...], axis=1, keepdims=True) + rb2[0]


def _tc_mlp(a0, a1, w1at, w1bt, rb1, rw2, rb2):
    blk = 512
    return pl.pallas_call(
        _mlp_body,
        grid=(_P // blk,),
        in_specs=[pl.BlockSpec((blk, _D), lambda i: (i, 0)),
                  pl.BlockSpec((blk, _D), lambda i: (i, 0)),
                  pl.BlockSpec((_D, _DH), lambda i: (0, 0)),
                  pl.BlockSpec((_D, _DH), lambda i: (0, 0)),
                  pl.BlockSpec((_DH,), lambda i: (0,)),
                  pl.BlockSpec((1, _DH), lambda i: (0, 0)),
                  pl.BlockSpec(memory_space=pltpu.SMEM)],
        out_specs=pl.BlockSpec((blk, 1), lambda i: (i, 0)),
        out_shape=jax.ShapeDtypeStruct((_P, 1), _f32),
    )(a0, a1, w1at, w1bt, rb1, rw2, rb2)


def kernel(drug_drug_batch, x, edge_index, batch,
           W1, b1, W2, b2, RW1, Rb1, RW2, Rb2):
    src, dst = edge_index[0], edge_index[1]
    pad_e = _EPAD - _E
    fill = jnp.full((pad_e,), _N, _i32)
    srcp = jnp.concatenate([src, fill]).reshape(_NS, _NC, _NCHUNK, _CH)
    dstp = jnp.concatenate([dst, fill]).reshape(_NS, _NC, _NCHUNK, _CH)
    xp = jnp.pad(x, ((0, _NP - _N), (0, 0)))
    batch2d = jnp.concatenate(
        [batch, jnp.full((_NP - _N,), _G + 5, _i32)]).reshape(_NBLK, 1, _RPT)
    flat = drug_drug_batch.reshape(2 * _P)
    w1t, w2t = W1.T, W2.T
    w1at, w1bt = RW1[:, :_D].T, RW1[:, _D:].T

    deg_part = _sc_hist(dstp.reshape(_NW, _NCHUNK, _CH)).T
    pres_row = _tc_pres(flat.reshape(8, 1024, 1))
    qtz = _tc_qtz(flat.reshape(8, 1, 1024), pres_row)
    h1 = _tc_matmul(xp, w1t)
    g1 = _tc_scale(h1, deg_part)
    part1 = _sc_scatter(g1, srcp, dstp)
    g2 = _tc_combine1(part1, deg_part, g1, b1, w2t)
    part2 = _sc_scatter(g2, srcp, dstp)
    ge = _tc_combine2(part2, deg_part, g2, b2, batch2d)
    cat2 = _sc_pair(ge, qtz.reshape(_NW, 2, _CH))
    return _tc_mlp(cat2[:_P], cat2[_P:], w1at, w1bt, Rb1, RW2, Rb2)


# speedup vs baseline: 10.5882x; 1.2033x over previous
import functools

import jax
import jax.numpy as jnp
from jax import lax
from jax.experimental import pallas as pl
from jax.experimental.pallas import tpu as pltpu
from jax.experimental.pallas import tpu_sc as plsc

_N = 10000
_E = 320000
_G = 500
_D = 128
_DH = 256
_P = 4096

_NC, _NS, _L = 2, 16, 16
_NW = _NC * _NS
_NP = 10240
_RPT = _NP // _NS
_CH = 128
_NCHUNK = 79
_SCHUNK = 158
_EPAD = _CH * _NCHUNK * _NW
_NH = _NP // 2
_ACCR = 5248
_GP = 512
_NBUF = 3
_PPT = (2 * _P) // _NW
_NBLK = 16

_mesh = plsc.VectorSubcoreMesh(
    core_axis_name="c", subcore_axis_name="s", num_cores=_NC, num_subcores=_NS)

_f32 = jnp.float32
_i32 = jnp.int32


def _hist_body(dst_hbm, deg_out, eidx, hist):
    cid = lax.axis_index("c")
    sid = lax.axis_index("s")
    wid = sid * _NC + cid
    pltpu.sync_copy(dst_hbm.at[wid], eidx)

    zero16 = jnp.zeros((_L,), _f32)

    def _z(i, _):
        hist[pl.ds(i * _L, _L)] = zero16
        return 0
    lax.fori_loop(0, _NP // _L, _z, 0)

    one16 = jnp.ones((_L,), _f32)

    def _edge(c, _):
        for j in range(_CH // _L):
            ids = eidx[c, pl.ds(j * _L, _L)]
            plsc.addupdate_scatter(hist, [ids], one16)
        return 0
    lax.fori_loop(0, _NCHUNK, _edge, 0)
    pltpu.sync_copy(hist, deg_out.at[wid])


_sc_hist = functools.partial(
    pl.kernel,
    out_type=jax.ShapeDtypeStruct((_NW, _NP), _f32),
    mesh=_mesh,
    compiler_params=pltpu.CompilerParams(needs_layout_passes=False),
    scratch_types=[pltpu.VMEM((_NCHUNK, _CH), _i32),
                   pltpu.VMEM((_NP,), _f32)],
)(_hist_body)


def _scatter_body(g_hbm, src_hbm, dst_hbm, part_out,
                  sidx, didx, pflat, rows, acc, semg, sems):
    cid = lax.axis_index("c")
    sid = lax.axis_index("s")
    base = cid * _NH
    zero16 = jnp.zeros((_L,), _f32)

    def _zrow(i, _):
        for j in range(_D // _L):
            rows[0, i, pl.ds(j * _L, _L)] = zero16
        return 0
    lax.fori_loop(0, _CH, _zrow, 0)
    zslab = _ACCR // _NS
    pltpu.sync_copy(rows.at[0], acc.at[pl.ds(sid * zslab, _CH)])
    pltpu.sync_copy(rows.at[0], acc.at[pl.ds(sid * zslab + _CH, _CH)])
    pltpu.sync_copy(rows.at[0, pl.ds(0, zslab - 2 * _CH)],
                    acc.at[pl.ds(sid * zslab + 2 * _CH, zslab - 2 * _CH)])
    plsc.subcore_barrier()

    for h in range(2):
        hh = jnp.int32(h) + 0 * cid
        pltpu.sync_copy(src_hbm.at[sid, hh], sidx)
        pltpu.sync_copy(dst_hbm.at[sid, hh], didx)

        def _cmp(c, off):
            for j in range(_CH // _L):
                sl = pl.ds(j * _L, _L)
                d = didx[c, sl] - base
                ok = (d >= 0) & (d < _NH)
                packed = d * 16384 + sidx[c, sl]
                plsc.store_compressed(pflat.at[pl.ds(off, _L)], packed,
                                      mask=ok)
                off = off + jnp.sum(ok.astype(_i32))
            return off
        off = lax.fori_loop(0, _NCHUNK, _cmp, jnp.int32(0))

        junk_p = jnp.full((_L,), _NH * 16384 + _N, _i32)
        for k in range(2 * _CH // _L):
            pflat[pl.ds(off + k * _L, _L)] = junk_p
        nch = jnp.maximum((off + _CH - 1) // _CH, 2)

        def _cpy(r, _):
            for j in range(_CH // _L):
                sl = pl.ds(j * _L, _L)
                v = pflat[pl.ds(r * _CH + j * _L, _L)]
                sidx[r, sl] = jnp.bitwise_and(v, 16383)
                didx[r, sl] = lax.shift_right_logical(v, 14)
            return 0
        lax.fori_loop(0, nch, _cpy, 0)

        pltpu.async_copy(g_hbm.at[sidx.at[0]], rows.at[0], semg)
        pltpu.async_copy(g_hbm.at[sidx.at[1]], rows.at[1], semg)

        def _step(c, _):
            b = lax.rem(c, _NBUF)
            pltpu.make_async_copy(g_hbm.at[pl.ds(0, _CH)], rows.at[b],
                                  semg).wait()
            pltpu.async_copy(rows.at[b], acc.at[didx.at[c]], sems, add=True)
            @pl.when(c >= 1)
            def _():
                bo = lax.rem(c - 1, _NBUF)
                pltpu.make_async_copy(rows.at[bo], acc.at[pl.ds(0, _CH)],
                                      sems).wait()
            @pl.when(c + 2 < nch)
            def _():
                bn = lax.rem(c + 2, _NBUF)
                pltpu.async_copy(g_hbm.at[sidx.at[c + 2]], rows.at[bn], semg)
            return 0
        lax.fori_loop(0, nch, _step, 0)
        pltpu.make_async_copy(rows.at[lax.rem(nch - 1, _NBUF)],
                              acc.at[pl.ds(0, _CH)], sems).wait()
    plsc.subcore_barrier()

    slab = _NH // _NS
    pltpu.sync_copy(acc.at[pl.ds(sid * slab, slab)],
                    part_out.at[pl.ds(cid * _NH + sid * slab, slab)])


_sc_scatter = functools.partial(
    pl.kernel,
    out_type=jax.ShapeDtypeStruct((_NP, _D), _f32),
    mesh=_mesh,
    compiler_params=pltpu.CompilerParams(needs_layout_passes=False),
    scratch_types=[pltpu.VMEM((_NCHUNK, _CH), _i32),
                   pltpu.VMEM((_NCHUNK, _CH), _i32),
                   pltpu.VMEM((_NCHUNK * _CH + 2 * _CH,), _i32),
                   pltpu.VMEM((_NBUF, _CH, _D), _f32),
                   pltpu.VMEM_SHARED((_ACCR, _D), _f32),
                   pltpu.SemaphoreType.DMA,
                   pltpu.SemaphoreType.DMA],
)(_scatter_body)


def _pair_body(ge_hbm, qtz_hbm, cat_out, qidx, rows, sem):
    cid = lax.axis_index("c")
    sid = lax.axis_index("s")
    wid = sid * _NC + cid
    pltpu.sync_copy(qtz_hbm.at[wid], qidx)
    for half in range(2):
        pltpu.async_copy(ge_hbm.at[qidx.at[half]],
                         rows.at[pl.ds(half * _CH, _CH)], sem).wait()
    pltpu.sync_copy(rows, cat_out.at[pl.ds(wid * _PPT, _PPT)])


_sc_pair = functools.partial(
    pl.kernel,
    out_type=jax.ShapeDtypeStruct((2 * _P, _D), _f32),
    mesh=_mesh,
    scratch_types=[pltpu.VMEM((2, _CH), _i32),
                   pltpu.VMEM((_PPT, _D), _f32),
                   pltpu.SemaphoreType.DMA],
)(_pair_body)


def _dinv_of(deg_blk):
    d = 1.0 + jnp.sum(deg_blk, axis=1, keepdims=True)
    return lax.rsqrt(jnp.maximum(d, 1.0))


def _mm_body(x_ref, w_ref, o_ref):
    o_ref[...] = jnp.dot(x_ref[...], w_ref[...],
                         preferred_element_type=_f32,
                         precision=lax.Precision.HIGHEST)


def _tc_matmul(xp, w):
    return pl.pallas_call(
        _mm_body,
        grid=(_NBLK,),
        in_specs=[pl.BlockSpec((_RPT, _D), lambda i: (i, 0)),
                  pl.BlockSpec((_D, _D), lambda i: (0, 0))],
        out_specs=pl.BlockSpec((_RPT, _D), lambda i: (i, 0)),
        out_shape=jax.ShapeDtypeStruct((_NP, _D), _f32),
    )(xp, w)


def _scale_body(h_ref, deg_ref, g_ref):
    dinv = _dinv_of(deg_ref[...])
    g_ref[...] = h_ref[...] * dinv


def _tc_scale(h1, deg_part):
    return pl.pallas_call(
        _scale_body,
        grid=(_NBLK,),
        in_specs=[pl.BlockSpec((_RPT, _D), lambda i: (i, 0)),
                  pl.BlockSpec((_RPT, _NW), lambda i: (i, 0))],
        out_specs=pl.BlockSpec((_RPT, _D), lambda i: (i, 0)),
        out_shape=jax.ShapeDtypeStruct((_NP, _D), _f32),
    )(h1, deg_part)


def _pres_body(d_ref, pres_ref, acc):
    i = pl.program_id(0)
    @pl.when(i == 0)
    def _():
        acc[...] = jnp.zeros_like(acc)
    d = d_ref[...][0]
    gids = lax.broadcasted_iota(_i32, (1024, _GP), 1)
    oh = (gids == d).astype(_f32)
    acc[...] += jnp.dot(jnp.ones((1, 1024), _f32), oh,
                        preferred_element_type=_f32,
                        precision=lax.Precision.HIGHEST)
    @pl.when(i == 7)
    def _():
        pres_ref[...] = jnp.minimum(acc[...], 1.0)


def _tc_pres(ddb3):
    return pl.pallas_call(
        _pres_body,
        grid=(8,),
        in_specs=[pl.BlockSpec((1, 1024, 1), lambda i: (i, 0, 0))],
        out_specs=pl.BlockSpec((1, _GP), lambda i: (0, 0)),
        out_shape=jax.ShapeDtypeStruct((1, _GP), _f32),
        scratch_shapes=[pltpu.VMEM((1, _GP), _f32)],
    )(ddb3)


def _qtz_body(d_ref, pres_ref, q_ref):
    d = d_ref[...][0]
    gids = lax.broadcasted_iota(_i32, (_GP, 1024), 0)
    lt = (gids < d).astype(_f32)
    q = jnp.dot(pres_ref[...], lt, preferred_element_type=_f32,
                precision=lax.Precision.HIGHEST)
    q_ref[...] = jnp.round(q).astype(_i32).reshape(1, 1, 1024)


def _tc_qtz(ddb2, pres_row):
    return pl.pallas_call(
        _qtz_body,
        grid=(8,),
        in_specs=[pl.BlockSpec((1, 1, 1024), lambda i: (i, 0, 0)),
                  pl.BlockSpec((1, _GP), lambda i: (0, 0))],
        out_specs=pl.BlockSpec((1, 1, 1024), lambda i: (i, 0, 0)),
        out_shape=jax.ShapeDtypeStruct((8, 1, 1024), _i32),
    )(ddb2, pres_row)


def _comb1_body(p_ref, deg_ref, g_ref, b_ref, w_ref, o_ref):
    dinv = _dinv_of(deg_ref[...])
    t = dinv * (p_ref[...] + g_ref[...]) + b_ref[...]
    t = jnp.maximum(t, 0.0)
    ridx = (pl.program_id(0) * _RPT
            + lax.broadcasted_iota(_i32, (_RPT, 1), 0))
    vmask = (ridx < _N).astype(_f32)
    o_ref[...] = vmask * dinv * jnp.dot(t, w_ref[...],
                                        preferred_element_type=_f32,
                                        precision=lax.Precision.HIGHEST)


def _tc_combine1(part, deg_part, g1, b1, w2t):
    return pl.pallas_call(
        _comb1_body,
        grid=(_NBLK,),
        in_specs=[pl.BlockSpec((_RPT, _D), lambda i: (i, 0)),
                  pl.BlockSpec((_RPT, _NW), lambda i: (i, 0)),
                  pl.BlockSpec((_RPT, _D), lambda i: (i, 0)),
                  pl.BlockSpec((_D,), lambda i: (0,)),
                  pl.BlockSpec((_D, _D), lambda i: (0, 0))],
        out_specs=pl.BlockSpec((_RPT, _D), lambda i: (i, 0)),
        out_shape=jax.ShapeDtypeStruct((_NP, _D), _f32),
    )(part, deg_part, g1, b1, w2t)


def _comb2_body(p_ref, deg_ref, g_ref, b_ref, bat_ref, ge_ref, sums, cnt):
    i = pl.program_id(0)
    @pl.when(i == 0)
    def _():
        sums[...] = jnp.zeros_like(sums)
        cnt[...] = jnp.zeros_like(cnt)
    dinv = _dinv_of(deg_ref[...])
    t = jnp.maximum(dinv * (p_ref[...] + g_ref[...]) + b_ref[...], 0.0)
    bat = bat_ref[...][0]
    gr = lax.broadcasted_iota(_i32, (_GP, _RPT), 0)
    oht = (gr == bat).astype(_f32)
    hi = lax.Precision.HIGHEST
    sums[...] += jnp.dot(oht, t, preferred_element_type=_f32, precision=hi)
    cnt[...] += jnp.dot(oht, jnp.ones((_RPT, 1), _f32),
                        preferred_element_type=_f32, precision=hi)
    @pl.when(i == _NBLK - 1)
    def _():
        ge_ref[...] = sums[...] / jnp.maximum(cnt[...], 1.0)


def _tc_combine2(part, deg_part, g2, b2, batch2d):
    return pl.pallas_call(
        _comb2_body,
        grid=(_NBLK,),
        in_specs=[pl.BlockSpec((_RPT, _D), lambda i: (i, 0)),
                  pl.BlockSpec((_RPT, _NW), lambda i: (i, 0)),
                  pl.BlockSpec((_RPT, _D), lambda i: (i, 0)),
                  pl.BlockSpec((_D,), lambda i: (0,)),
                  pl.BlockSpec((1, 1, _RPT), lambda i: (i, 0, 0))],
        out_specs=pl.BlockSpec((_GP, _D), lambda i: (0, 0)),
        out_shape=jax.ShapeDtypeStruct((_GP, _D), _f32),
        scratch_shapes=[pltpu.VMEM((_GP, _D), _f32),
                        pltpu.VMEM((_GP, 1), _f32)],
    )(part, deg_part, g2, b2, batch2d)


def _mlp_body(a0, a1, wa, wb, rb1, rw2, rb2, o_ref):
    hi = lax.Precision.HIGHEST
    z = (jnp.dot(a0[...], wa[...], preferred_element_type=_f32, precision=hi)
         + jnp.dot(a1[...], wb[...], preferred_element_type=_f32, precision=hi)
         + rb1[...])
    z = jnp.maximum(z, 0.0)
    o_ref[...] = jnp.sum(z * rw2[...], axis=1, keepdims=True) + rb2[0]


def _tc_mlp(a0, a1, w1at, w1bt, rb1, rw2, rb2):
    blk = 512
    return pl.pallas_call(
        _mlp_body,
        grid=(_P // blk,),
        in_specs=[pl.BlockSpec((blk, _D), lambda i: (i, 0)),
                  pl.BlockSpec((blk, _D), lambda i: (i, 0)),
                  pl.BlockSpec((_D, _DH), lambda i: (0, 0)),
                  pl.BlockSpec((_D, _DH), lambda i: (0, 0)),
                  pl.BlockSpec((_DH,), lambda i: (0,)),
                  pl.BlockSpec((1, _DH), lambda i: (0, 0)),
                  pl.BlockSpec(memory_space=pltpu.SMEM)],
        out_specs=pl.BlockSpec((blk, 1), lambda i: (i, 0)),
        out_shape=jax.ShapeDtypeStruct((_P, 1), _f32),
    )(a0, a1, w1at, w1bt, rb1, rw2, rb2)


def kernel(drug_drug_batch, x, edge_index, batch,
           W1, b1, W2, b2, RW1, Rb1, RW2, Rb2):
    src, dst = edge_index[0], edge_index[1]
    pad_e = _EPAD - _E
    fill = jnp.full((pad_e,), _N, _i32)
    srcp = jnp.concatenate([src, fill]).reshape(_NS, _NC, _NCHUNK, _CH)
    dstp = jnp.concatenate([dst, fill]).reshape(_NS, _NC, _NCHUNK, _CH)
    xp = jnp.pad(x, ((0, _NP - _N), (0, 0)))
    batch2d = jnp.concatenate(
        [batch, jnp.full((_NP - _N,), _G + 5, _i32)]).reshape(_NBLK, 1, _RPT)
    flat = drug_drug_batch.reshape(2 * _P)
    w1t, w2t = W1.T, W2.T
    w1at, w1bt = RW1[:, :_D].T, RW1[:, _D:].T

    deg_part = _sc_hist(dstp.reshape(_NW, _NCHUNK, _CH)).T
    pres_row = _tc_pres(flat.reshape(8, 1024, 1))
    qtz = _tc_qtz(flat.reshape(8, 1, 1024), pres_row)
    h1 = _tc_matmul(xp, w1t)
    g1 = _tc_scale(h1, deg_part)
    part1 = _sc_scatter(g1, srcp, dstp)
    g2 = _tc_combine1(part1, deg_part, g1, b1, w2t)
    part2 = _sc_scatter(g2, srcp, dstp)
    ge = _tc_combine2(part2, deg_part, g2, b2, batch2d)
    cat2 = _sc_pair(ge, qtz.reshape(_NW, 2, _CH))
    return _tc_mlp(cat2[:_P], cat2[_P:], w1at, w1bt, Rb1, RW2, Rb2)
